# initial kernel scaffold (unmeasured)
import jax
import jax.numpy as jnp
from jax import lax
from jax.experimental import pallas as pl
from jax.experimental.pallas import tpu as pltpu

N_DEV = 8
M = 8192
M_PER = M // N_DEV


def _gemm_body(x_ref, w_ref, out_ref):
    x = x_ref[...].astype(jnp.bfloat16)
    w = w_ref[...].astype(jnp.bfloat16)
    acc = jnp.dot(x, w, preferred_element_type=jnp.float32)
    out_ref[...] = acc.astype(jnp.bfloat16)


def _local_gemm(x, w):
    m, k = x.shape
    _, n = w.shape
    bm, bn = 1024, 1024
    return pl.pallas_call(
        _gemm_body,
        grid=(m // bm, n // bn),
        in_specs=[
            pl.BlockSpec((bm, k), lambda mi, ni: (mi, 0)),
            pl.BlockSpec((k, bn), lambda mi, ni: (0, ni)),
        ],
        out_specs=pl.BlockSpec((bm, bn), lambda mi, ni: (mi, ni)),
        out_shape=jax.ShapeDtypeStruct((m, n), jnp.bfloat16),
    )(x, w)


def _rs_body(partial_ref, out_ref, comm_ref, local_ref,
             send_sems, recv_sems, local_sem):
    my = lax.axis_index("i")
    left = lax.rem(my + N_DEV - 1, N_DEV)
    right = lax.rem(my + 1, N_DEV)

    barrier = pltpu.get_barrier_semaphore()
    for nbr in (left, right):
        pl.semaphore_signal(barrier, inc=1, device_id=(nbr,),
                            device_id_type=pl.DeviceIdType.MESH)
    pl.semaphore_wait(barrier, 2)

    def dma_local_chunk(c, dst):
        cp = pltpu.make_async_copy(
            partial_ref.at[pl.ds(c * M_PER, M_PER), :], dst, local_sem)
        cp.start()
        cp.wait()

    dma_local_chunk(lax.rem(my + N_DEV - 1, N_DEV), comm_ref.at[0])

    for s in range(N_DEV - 1):
        send_slot = s % 2
        recv_slot = (s + 1) % 2
        rdma = pltpu.make_async_remote_copy(
            src_ref=comm_ref.at[send_slot],
            dst_ref=comm_ref.at[recv_slot],
            send_sem=send_sems.at[send_slot],
            recv_sem=recv_sems.at[recv_slot],
            device_id=(right,),
            device_id_type=pl.DeviceIdType.MESH,
        )
        rdma.start()
        rdma.wait()

        c = lax.rem(my + 2 * N_DEV - 2 - s, N_DEV)
        dma_local_chunk(c, local_ref)
        if s < N_DEV - 2:
            comm_ref[recv_slot] = (comm_ref[recv_slot] + local_ref[...])
        else:
            y = (comm_ref[recv_slot].astype(jnp.float32)
                 + local_ref[...].astype(jnp.float32))
            out_ref[...] = y * jax.nn.sigmoid(y)


def _reduce_scatter_silu(partial):
    _, n = partial.shape
    return pl.pallas_call(
        _rs_body,
        out_shape=jax.ShapeDtypeStruct((M_PER, n), jnp.float32),
        in_specs=[pl.BlockSpec(memory_space=pltpu.ANY)],
        out_specs=pl.BlockSpec(memory_space=pltpu.VMEM),
        scratch_shapes=[
            pltpu.VMEM((2, M_PER, n), jnp.bfloat16),
            pltpu.VMEM((M_PER, n), jnp.bfloat16),
            pltpu.SemaphoreType.DMA((2,)),
            pltpu.SemaphoreType.DMA((2,)),
            pltpu.SemaphoreType.DMA,
        ],
        compiler_params=pltpu.CompilerParams(collective_id=0),
    )(partial)


def kernel(x, w_mat):
    partial = _local_gemm(x, w_mat)
    return _reduce_scatter_silu(partial)


# baseline (device time: 784679 ns/iter reference)
import jax
import jax.numpy as jnp
from jax import lax
from jax.experimental import pallas as pl
from jax.experimental.pallas import tpu as pltpu

N_DEV = 8
M = 8192
M_PER = M // N_DEV


def _gemm_body(x_ref, w_ref, out_ref):
    x = x_ref[...].astype(jnp.bfloat16)
    w = w_ref[...].astype(jnp.bfloat16)
    acc = jnp.dot(x, w, preferred_element_type=jnp.float32)
    out_ref[...] = acc.astype(jnp.bfloat16)


def _local_gemm(x, w):
    m, k = x.shape
    _, n = w.shape
    bm, bn = 1024, 1024
    return pl.pallas_call(
        _gemm_body,
        grid=(m // bm, n // bn),
        in_specs=[
            pl.BlockSpec((bm, k), lambda mi, ni: (mi, 0)),
            pl.BlockSpec((k, bn), lambda mi, ni: (0, ni)),
        ],
        out_specs=pl.BlockSpec((bm, bn), lambda mi, ni: (mi, ni)),
        out_shape=jax.ShapeDtypeStruct((m, n), jnp.bfloat16),
    )(x, w)


def _rs_body(partial_ref, out_ref, comm_ref, local_ref,
             send_sems, recv_sems, local_sem):
    my = lax.axis_index("i")
    left = lax.rem(my + N_DEV - 1, N_DEV)
    right = lax.rem(my + 1, N_DEV)

    barrier = pltpu.get_barrier_semaphore()
    for nbr in (left, right):
        pl.semaphore_signal(barrier, inc=1, device_id=(nbr,),
                            device_id_type=pl.DeviceIdType.MESH)
    pl.semaphore_wait(barrier, 2)

    def dma_local_chunk(c, dst):
        cp = pltpu.make_async_copy(
            partial_ref.at[pl.ds(c * M_PER, M_PER), :], dst, local_sem)
        cp.start()
        cp.wait()

    dma_local_chunk(lax.rem(my + N_DEV - 1, N_DEV), comm_ref.at[0])

    for s in range(N_DEV - 1):
        send_slot = s % 2
        recv_slot = (s + 1) % 2
        rdma = pltpu.make_async_remote_copy(
            src_ref=comm_ref.at[send_slot],
            dst_ref=comm_ref.at[recv_slot],
            send_sem=send_sems.at[send_slot],
            recv_sem=recv_sems.at[recv_slot],
            device_id=(right,),
            device_id_type=pl.DeviceIdType.MESH,
        )
        rdma.start()
        rdma.wait()

        c = lax.rem(my + 2 * N_DEV - 2 - s, N_DEV)
        dma_local_chunk(c, local_ref)
        if s < N_DEV - 2:
            comm_ref[recv_slot] = (comm_ref[recv_slot] + local_ref[...])
        else:
            y = (comm_ref[recv_slot].astype(jnp.float32)
                 + local_ref[...].astype(jnp.float32))
            out_ref[...] = y * jax.nn.sigmoid(y)


def _reduce_scatter_silu(partial):
    _, n = partial.shape
    return pl.pallas_call(
        _rs_body,
        out_shape=jax.ShapeDtypeStruct((M_PER, n), jnp.float32),
        in_specs=[pl.BlockSpec(memory_space=pl.ANY)],
        out_specs=pl.BlockSpec(memory_space=pltpu.VMEM),
        scratch_shapes=[
            pltpu.VMEM((2, M_PER, n), jnp.bfloat16),
            pltpu.VMEM((M_PER, n), jnp.bfloat16),
            pltpu.SemaphoreType.DMA((2,)),
            pltpu.SemaphoreType.DMA((2,)),
            pltpu.SemaphoreType.DMA,
        ],
        compiler_params=pltpu.CompilerParams(collective_id=0),
    )(partial)


def kernel(x, w_mat):
    partial = _local_gemm(x, w_mat)
    return _reduce_scatter_silu(partial)


# device time: 401535 ns/iter; 1.9542x vs baseline; 1.9542x over previous
import jax
import jax.numpy as jnp
from jax import lax
from jax.experimental import pallas as pl
from jax.experimental.pallas import tpu as pltpu

N_DEV = 8
M = 8192
M_PER = M // N_DEV
N_OUT = 4096
HALF = N_OUT // 2


def _fused_body(x_hbm, w_ref, out_ref, xa_ref, xb_ref, fwd_ref, rev_ref,
                scr_a, scr_b, xa_sem, xb_sem,
                fwd_send_sems, fwd_recv_sems, rev_send_sems, rev_recv_sems):
    my = lax.axis_index("i")
    left = lax.rem(my + N_DEV - 1, N_DEV)
    right = lax.rem(my + 1, N_DEV)

    barrier = pltpu.get_barrier_semaphore()
    for nbr in (left, right):
        pl.semaphore_signal(barrier, inc=1, device_id=(nbr,),
                            device_id_type=pl.DeviceIdType.MESH)
    pl.semaphore_wait(barrier, 2)

    def fetch(c, dst, sem):
        cp = pltpu.make_async_copy(
            x_hbm.at[pl.ds(c * M_PER, M_PER), :], dst, sem)
        cp.start()
        return cp

    def dot_a(xc):
        return jnp.dot(xc[...], w_ref[:, :HALF],
                       preferred_element_type=jnp.float32)

    def dot_b(xc):
        return jnp.dot(xc[...], w_ref[:, HALF:],
                       preferred_element_type=jnp.float32)

    fa = fetch(lax.rem(my + N_DEV - 1, N_DEV), xa_ref, xa_sem)
    fb = fetch(lax.rem(my + 1, N_DEV), xb_ref, xb_sem)
    fa.wait()
    fwd_ref[0] = dot_a(xa_ref).astype(jnp.bfloat16)
    fb.wait()
    rev_ref[0] = dot_b(xb_ref).astype(jnp.bfloat16)

    for s in range(N_DEV - 1):
        ss, rs = s % 2, (s + 1) % 2
        fwd = pltpu.make_async_remote_copy(
            src_ref=fwd_ref.at[ss], dst_ref=fwd_ref.at[rs],
            send_sem=fwd_send_sems.at[ss], recv_sem=fwd_recv_sems.at[rs],
            device_id=(right,), device_id_type=pl.DeviceIdType.MESH)
        rev = pltpu.make_async_remote_copy(
            src_ref=rev_ref.at[ss], dst_ref=rev_ref.at[rs],
            send_sem=rev_send_sems.at[ss], recv_sem=rev_recv_sems.at[rs],
            device_id=(left,), device_id_type=pl.DeviceIdType.MESH)
        fwd.start()
        rev.start()
        ca = lax.rem(my + 2 * N_DEV - 2 - s, N_DEV)
        cb = lax.rem(my + 2 + s, N_DEV)
        fa = fetch(ca, xa_ref, xa_sem)
        fb = fetch(cb, xb_ref, xb_sem)
        if s < N_DEV - 2:
            fa.wait()
            scr_a[...] = dot_a(xa_ref).astype(jnp.bfloat16)
            fb.wait()
            scr_b[...] = dot_b(xb_ref).astype(jnp.bfloat16)
            fwd.wait()
            fwd_ref[rs] = fwd_ref[rs] + scr_a[...]
            rev.wait()
            rev_ref[rs] = rev_ref[rs] + scr_b[...]
        else:
            fa.wait()
            a = dot_a(xa_ref)
            fwd.wait()
            ya = fwd_ref[rs].astype(jnp.float32) + a
            out_ref[:, :HALF] = (ya * jax.nn.sigmoid(ya)).astype(jnp.bfloat16)
            fb.wait()
            b = dot_b(xb_ref)
            rev.wait()
            yb = rev_ref[rs].astype(jnp.float32) + b
            out_ref[:, HALF:] = (yb * jax.nn.sigmoid(yb)).astype(jnp.bfloat16)


def kernel(x, w_mat):
    x = x.astype(jnp.bfloat16)
    w_mat = w_mat.astype(jnp.bfloat16)
    return pl.pallas_call(
        _fused_body,
        out_shape=jax.ShapeDtypeStruct((M_PER, N_OUT), jnp.bfloat16),
        in_specs=[
            pl.BlockSpec(memory_space=pl.ANY),
            pl.BlockSpec(memory_space=pltpu.VMEM),
        ],
        out_specs=pl.BlockSpec(memory_space=pltpu.VMEM),
        scratch_shapes=[
            pltpu.VMEM((M_PER, 1024), jnp.bfloat16),
            pltpu.VMEM((M_PER, 1024), jnp.bfloat16),
            pltpu.VMEM((2, M_PER, HALF), jnp.bfloat16),
            pltpu.VMEM((2, M_PER, HALF), jnp.bfloat16),
            pltpu.VMEM((M_PER, HALF), jnp.bfloat16),
            pltpu.VMEM((M_PER, HALF), jnp.bfloat16),
            pltpu.SemaphoreType.DMA,
            pltpu.SemaphoreType.DMA,
            pltpu.SemaphoreType.DMA((2,)),
            pltpu.SemaphoreType.DMA((2,)),
            pltpu.SemaphoreType.DMA((2,)),
            pltpu.SemaphoreType.DMA((2,)),
        ],
        compiler_params=pltpu.CompilerParams(
            collective_id=0, vmem_limit_bytes=128 * 1024 * 1024),
    )(x, w_mat)


# device time: 327476 ns/iter; 2.3961x vs baseline; 1.2262x over previous
import numpy as np

import jax
import jax.numpy as jnp
from jax import lax
from jax.experimental import pallas as pl
from jax.experimental.pallas import tpu as pltpu

N_DEV = 8
M = 8192
M_PER = M // N_DEV
N_OUT = 4096
W = 1408
HWF = 768
HWR = 640
N_PAD = 3 * W

_PARTNER = {
    0: [4, 5, 6, 7, 0, 1, 2, 3],
    1: [3, 2, 1, 0, 7, 6, 5, 4],
    2: [1, 0, 3, 2, 5, 4, 7, 6],
}
_FACES = {
    0: [[0, 1, 2, 3], [4, 5, 6, 7]],
    1: [[0, 1, 5, 4], [2, 3, 7, 6]],
    2: [[0, 3, 7, 4], [1, 2, 6, 5]],
}


def _chunk(g, d, j):
    for fc in _FACES[g]:
        if d in fc:
            q = fc.index(d)
            return fc[(q + j) % 4]
    raise AssertionError


def _build_table():
    rows = []
    for d in range(8):
        row = []
        for g in range(3):
            r = _PARTNER[g][d]
            row += [_chunk(g, r, 3), _chunk(g, r, 1),
                    _chunk(g, r, 2), _chunk(g, r, 0)]
            row += [_chunk(g, d, 3), _chunk(g, d, 1),
                    _chunk(g, d, 2), d]
            row += [_chunk(g, d, 1), _chunk(g, d, 3), r]
        rows.append(row)
    return np.array(rows, dtype=np.int32)


_TBL = _build_table()


def _body(tbl, x_hbm, w_ref, out_ref, hfold,
          xbuf, fsend, ringf, ringr, rb,
          x_sems, fsend_sems, frecv_sems,
          rf_send, rf_recv, rr_send, rr_recv, rb_sems, out_sems):
    my = lax.axis_index("i")

    def t(g, k):
        return tbl[my, g * 11 + k]

    barrier = pltpu.get_barrier_semaphore()
    for g in range(3):
        pl.semaphore_signal(barrier, inc=1, device_id=(t(g, 10),),
                            device_id_type=pl.DeviceIdType.MESH)
    pl.semaphore_wait(barrier, 3)

    def fetch_x(c, slot):
        cp = pltpu.make_async_copy(
            x_hbm.at[pl.ds(c * M_PER, M_PER), :], xbuf.at[slot],
            x_sems.at[slot])
        cp.start()
        return cp

    def gdot(slot, g):
        return jnp.dot(xbuf[slot], w_ref[:, g * W:(g + 1) * W],
                       preferred_element_type=jnp.float32)

    fold_rdmas = [None, None, None]
    fetches = [None] * 13
    fetches[0] = fetch_x(t(0, 0), 0)
    i = 0
    for k in range(4):
        for g in range(3):
            ng, nk = (g + 1, k) if g < 2 else (0, k + 1)
            if i < 11:
                fetches[i + 1] = fetch_x(t(ng, nk), (i + 1) % 2)
            fetches[i].wait()
            if fold_rdmas[g] is not None:
                fold_rdmas[g].wait_send()
            fsend[g] = gdot(i % 2, g).astype(jnp.bfloat16)
            rdma = pltpu.make_async_remote_copy(
                src_ref=fsend.at[g],
                dst_ref=hfold.at[k, :, pl.ds(g * W, W)],
                send_sem=fsend_sems.at[g],
                recv_sem=frecv_sems.at[g, k],
                device_id=(t(g, 10),),
                device_id_type=pl.DeviceIdType.MESH)
            rdma.start()
            fold_rdmas[g] = rdma
            i += 1

    def frecv_wait(g, k):
        pltpu.make_async_remote_copy(
            src_ref=fsend.at[g], dst_ref=hfold.at[k, :, pl.ds(g * W, W)],
            send_sem=fsend_sems.at[g], recv_sem=frecv_sems.at[g, k],
            device_id=(t(g, 10),),
            device_id_type=pl.DeviceIdType.MESH).wait_recv()

    rb_i = [0]

    def readback(g, k, col0, width):
        slot = rb_i[0] % 2
        rb_i[0] += 1
        cp = pltpu.make_async_copy(
            hfold.at[k, :, pl.ds(g * W + col0, width)],
            rb.at[slot, :, pl.ds(col0, width)], rb_sems.at[slot])
        cp.start()
        cp.wait()
        return slot

    for g in range(3):
        frecv_wait(g, 0)
        s_rb = readback(g, 0, 0, HWF)
        fetch_x(t(g, 4), 0).wait()
        ringf[g, 0] = (gdot(0, g)[:, :HWF]
                       + rb[s_rb, :, :HWF].astype(jnp.float32)
                       ).astype(jnp.bfloat16)
    for g in range(3):
        frecv_wait(g, 1)
        s_rb = readback(g, 1, HWF, HWR)
        fetch_x(t(g, 5), 0).wait()
        ringr[g, 0] = (gdot(0, g)[:, HWF:]
                       + rb[s_rb, :, HWF:].astype(jnp.float32)
                       ).astype(jnp.bfloat16)

    for s in range(3):
        ss, rs = s % 2, (s + 1) % 2
        frdmas, rrdmas = [], []
        for g in range(3):
            f = pltpu.make_async_remote_copy(
                src_ref=ringf.at[g, ss], dst_ref=ringf.at[g, rs],
                send_sem=rf_send.at[g, ss], recv_sem=rf_recv.at[g, rs],
                device_id=(t(g, 8),), device_id_type=pl.DeviceIdType.MESH)
            r = pltpu.make_async_remote_copy(
                src_ref=ringr.at[g, ss], dst_ref=ringr.at[g, rs],
                send_sem=rr_send.at[g, ss], recv_sem=rr_recv.at[g, rs],
                device_id=(t(g, 9),), device_id_type=pl.DeviceIdType.MESH)
            f.start()
            r.start()
            frdmas.append(f)
            rrdmas.append(r)
        if s == 0:
            for g in range(3):
                frecv_wait(g, 2)
                s_rb = readback(g, 2, 0, W)
                fetch_x(t(g, 6), 0).wait()
                d = gdot(0, g) + rb[s_rb].astype(jnp.float32)
                frdmas[g].wait()
                ringf[g, rs] = (ringf[g, rs].astype(jnp.float32)
                                + d[:, :HWF]).astype(jnp.bfloat16)
                rrdmas[g].wait()
                ringr[g, rs] = (ringr[g, rs].astype(jnp.float32)
                                + d[:, HWF:]).astype(jnp.bfloat16)
        elif s == 1:
            for g in range(3):
                s_rb1 = readback(g, 1, 0, HWF)
                fetch_x(t(g, 5), 0).wait()
                df = (gdot(0, g)[:, :HWF]
                      + rb[s_rb1, :, :HWF].astype(jnp.float32))
                s_rb0 = readback(g, 0, HWF, HWR)
                fetch_x(t(g, 4), 0).wait()
                dr = (gdot(0, g)[:, HWF:]
                      + rb[s_rb0, :, HWF:].astype(jnp.float32))
                frdmas[g].wait()
                ringf[g, rs] = (ringf[g, rs].astype(jnp.float32)
                                + df).astype(jnp.bfloat16)
                rrdmas[g].wait()
                ringr[g, rs] = (ringr[g, rs].astype(jnp.float32)
                                + dr).astype(jnp.bfloat16)
        else:
            fetch_x(my, 0).wait()
            for g in range(3):
                frecv_wait(g, 3)
                s_rb = readback(g, 3, 0, W)
                d = gdot(0, g) + rb[s_rb].astype(jnp.float32)
                fold_rdmas[g].wait_send()
                frdmas[g].wait()
                yf = ringf[g, rs].astype(jnp.float32) + d[:, :HWF]
                fsend[g, :, :HWF] = (yf * jax.nn.sigmoid(yf)
                                     ).astype(jnp.bfloat16)
                rrdmas[g].wait()
                yr = ringr[g, rs].astype(jnp.float32) + d[:, HWF:]
                fsend[g, :, HWF:] = (yr * jax.nn.sigmoid(yr)
                                     ).astype(jnp.bfloat16)

    outs = []
    for g in range(3):
        width = W if g < 2 else N_OUT - 2 * W
        cp = pltpu.make_async_copy(
            fsend.at[g, :, pl.ds(0, width)],
            out_ref.at[:, pl.ds(g * W, width)], out_sems.at[g])
        cp.start()
        outs.append(cp)
    for cp in outs:
        cp.wait()


def kernel(x, w_mat):
    x = x.astype(jnp.bfloat16)
    w_pad = jnp.pad(w_mat.astype(jnp.bfloat16), ((0, 0), (0, N_PAD - N_OUT)))
    out, _ = pl.pallas_call(
        _body,
        out_shape=(
            jax.ShapeDtypeStruct((M_PER, N_OUT), jnp.bfloat16),
            jax.ShapeDtypeStruct((4, M_PER, N_PAD), jnp.bfloat16),
        ),
        in_specs=[
            pl.BlockSpec(memory_space=pltpu.SMEM),
            pl.BlockSpec(memory_space=pl.ANY),
            pl.BlockSpec(memory_space=pltpu.VMEM),
        ],
        out_specs=(
            pl.BlockSpec(memory_space=pl.ANY),
            pl.BlockSpec(memory_space=pl.ANY),
        ),
        scratch_shapes=[
            pltpu.VMEM((2, M_PER, 1024), jnp.bfloat16),
            pltpu.VMEM((3, M_PER, W), jnp.bfloat16),
            pltpu.VMEM((3, 2, M_PER, HWF), jnp.bfloat16),
            pltpu.VMEM((3, 2, M_PER, HWR), jnp.bfloat16),
            pltpu.VMEM((2, M_PER, W), jnp.bfloat16),
            pltpu.SemaphoreType.DMA((2,)),
            pltpu.SemaphoreType.DMA((3,)),
            pltpu.SemaphoreType.DMA((3, 4)),
            pltpu.SemaphoreType.DMA((3, 2)),
            pltpu.SemaphoreType.DMA((3, 2)),
            pltpu.SemaphoreType.DMA((3, 2)),
            pltpu.SemaphoreType.DMA((3, 2)),
            pltpu.SemaphoreType.DMA((2,)),
            pltpu.SemaphoreType.DMA((3,)),
        ],
        compiler_params=pltpu.CompilerParams(
            collective_id=0, vmem_limit_bytes=128 * 1024 * 1024),
    )(jnp.asarray(_TBL), x, w_pad)
    return out
